# baseline (speedup 1.0000x reference)
import functools

import jax
import jax.numpy as jnp
from jax import lax
from jax.experimental import pallas as pl
from jax.experimental.pallas import tpu as pltpu
from jax.experimental.pallas import tpu_sc as plsc

GAMMA_ = 0.99
LANES = 16
N_CORES = 2
N_SUB = 16
SUB = 8
LN = 128
GRAN = 16


def _splat(vec, i):
    ci = jnp.full((LANES,), i, jnp.int32)
    return vec.at[ci].get(mode="promise_in_bounds")


def _lane_total(vec):
    return _splat(plsc.cumsum(vec), LANES - 1)


def _sc_body(consts, zpsi_hbm, ztgt_hbm, feat_hbm, act_hbm, sl_hbm,
             out_hbm, vact, vsl4, vfi, qp, qt, pdst, tdst, frow, part_ref,
             shared, gath, outv, semf, semg):
    B, L, n_gran, ipw = consts
    c = lax.axis_index("c")
    s = lax.axis_index("s")
    base = (c * N_SUB + s) * ipw
    b = lax.div(base, L)
    pltpu.sync_copy(act_hbm, vact)
    pltpu.sync_copy(sl_hbm, vsl4)
    lane = lax.iota(jnp.int32, LANES)
    t = jnp.minimum(lane + lax.rem(base, L), L - 1)
    bsp = lane * 0 + b
    av0 = plsc.load_gather(vact, [bsp, jnp.minimum(t, L - 2)])
    av1 = plsc.load_gather(vact, [bsp, jnp.minimum(t + 1, L - 2)])
    slb = plsc.load_gather(vsl4, [bsp])
    vfi[...] = b * (L + 1) + (t + 1)
    cpf = pltpu.async_copy(feat_hbm.at[vfi], frow, semf)
    sq0 = (base + lane) * (64 * 64) + lax.div(av0, LN) * 64 \
        + lax.div(lax.rem(av0, LN), GRAN)
    sq1 = (base + lane + 1) * (64 * 64) + lax.div(av1, LN) * 64 \
        + lax.div(lax.rem(av1, LN), GRAN)
    vlm0 = lax.rem(av0, GRAN)
    vlm1 = lax.rem(av1, GRAN)
    is_term = t == slb - 1
    cf_vec = jnp.where(is_term, 1.0, 0.0).astype(jnp.float32)
    cg_vec = jnp.where((t < L - 2) & jnp.logical_not(is_term),
                       GAMMA_, 0.0).astype(jnp.float32)
    vm_vec = jnp.where(t <= L - 2, 1.0, 0.0).astype(jnp.float32)
    offs = [lax.div(jnp.int32(16 * cc) + lane, SUB) * 512
            + lax.rem(jnp.int32(16 * cc) + lane, SUB) * 8
            for cc in range(4)]
    for i in range(ipw):
        sp0 = _splat(sq0, i)
        sp1 = _splat(sq1, i)
        for cc in range(4):
            qp[i, pl.ds(cc * LANES, LANES)] = sp0 + offs[cc]
            qt[i, pl.ds(cc * LANES, LANES)] = jnp.minimum(
                sp1 + offs[cc], n_gran - 1)
    cps = []
    for i in range(ipw):
        cps.append(pltpu.async_copy(zpsi_hbm.at[qp.at[i]],
                                    pdst.at[i], semg))
        cps.append(pltpu.async_copy(ztgt_hbm.at[qt.at[i]],
                                    tdst.at[i], semg))
    cpf.wait()
    for cp in cps:
        cp.wait()
    acc = jnp.zeros((LANES,), jnp.float32)
    for i in range(ipw):
        vls = _splat(vlm0, i)
        vlt = _splat(vlm1, i)
        cfs = _splat(cf_vec, i)
        cgs = _splat(cg_vec, i)
        vms = _splat(vm_vec, i)
        ii = jnp.full((LANES,), i, jnp.int32)
        for k in range(4):
            rows = lane + (k * LANES)
            pg = plsc.load_gather(pdst, [ii, rows, vls])
            tg = plsc.load_gather(tdst, [ii, rows, vlt])
            fr = frow[i, pl.ds(k * LANES, LANES)]
            d = vms * pg - cgs * tg - cfs * fr
            acc = acc + d * d
    part_ref[...] = acc
    pltpu.sync_copy(part_ref, shared.at[s])
    plsc.subcore_barrier()

    @pl.when(s == 0)
    def _reduce():
        pltpu.sync_copy(shared, gath)
        tot = gath[0, :]
        for i in range(1, N_SUB):
            tot = tot + gath[i, :]
        total = _lane_total(tot)
        slv = plsc.load_gather(vsl4, [jnp.minimum(lane, B - 1)])
        slm = jnp.where(lane < B, slv, 0).astype(jnp.float32)
        denom = _lane_total(slm)
        outv[...] = total / denom
        pltpu.sync_copy(outv, out_hbm.at[c])


def _combine_body(x_ref, o_ref):
    o_ref[...] = (x_ref[0, :] + x_ref[1, :]).reshape(1, LANES)


def kernel(psi, target_psi, actions, features, seq_lens):
    B, L, V, F = psi.shape
    n_gran = B * L * (F // SUB) * (V // LN) * SUB * (LN // GRAN)
    ipw = (B * L) // (N_CORES * N_SUB)
    assert (B * L) % (N_CORES * N_SUB) == 0 and L % ipw == 0 and ipw <= LANES
    assert F == 64 and V // LN == SUB

    def gran_view(x):
        return (x.reshape(B, L, V // LN, LN, F // SUB, SUB)
                 .transpose(0, 1, 4, 2, 5, 3)
                 .reshape(n_gran, GRAN))

    zpsi = gran_view(psi)
    ztgt = gran_view(target_psi)
    feat2 = features.reshape(B * (L + 1), F)
    act = actions.astype(jnp.int32)
    sl4 = seq_lens.astype(jnp.int32)

    mesh = plsc.VectorSubcoreMesh(
        core_axis_name="c", subcore_axis_name="s", num_cores=N_CORES)
    run = pl.kernel(
        functools.partial(_sc_body, (B, L, n_gran, ipw)),
        out_type=jax.ShapeDtypeStruct((N_CORES, LANES), jnp.float32),
        mesh=mesh,
        compiler_params=pltpu.CompilerParams(
            use_tc_tiling_on_sc=False, needs_layout_passes=False),
        scratch_types=[
            pltpu.VMEM((B, L - 1), jnp.int32),
            pltpu.VMEM((B,), jnp.int32),
            pltpu.VMEM((LANES,), jnp.int32),
            pltpu.VMEM((ipw, 64), jnp.int32),
            pltpu.VMEM((ipw, 64), jnp.int32),
            pltpu.VMEM((ipw, 64, GRAN), jnp.float32),
            pltpu.VMEM((ipw, 64, GRAN), jnp.float32),
            pltpu.VMEM((LANES, F), jnp.float32),
            pltpu.VMEM((LANES,), jnp.float32),
            pltpu.VMEM_SHARED((N_SUB, LANES), jnp.float32),
            pltpu.VMEM((N_SUB, LANES), jnp.float32),
            pltpu.VMEM((LANES,), jnp.float32),
            pltpu.SemaphoreType.DMA,
            pltpu.SemaphoreType.DMA,
        ],
    )
    parts = run(zpsi, ztgt, feat2, act, sl4)
    combine = pl.pallas_call(
        _combine_body,
        out_shape=jax.ShapeDtypeStruct((1, LANES), jnp.float32),
    )
    return combine(parts)[0, 0]


# speedup vs baseline: 1.1270x; 1.1270x over previous
import functools

import jax
import jax.numpy as jnp
from jax import lax
from jax.experimental import pallas as pl
from jax.experimental.pallas import tpu as pltpu
from jax.experimental.pallas import tpu_sc as plsc

GAMMA_ = 0.99
LANES = 16
N_WORKERS = 16
SUB = 8
LN = 128
GRAN = 16


def _splat(vec, i):
    ci = jnp.full((LANES,), i, jnp.int32)
    return vec.at[ci].get(mode="promise_in_bounds")


def _lane_total(vec):
    return _splat(plsc.cumsum(vec), LANES - 1)


def _sc_body(consts, zpsi_hbm, ztgt_hbm, feat_hbm, act_hbm, sl_hbm,
             out_hbm, vact, vsl4, vfi, qp, qt, pdst, tdst, frow, part_ref,
             shared, gath, outv, semf, semg):
    B, L, n_gran, ipw = consts
    s = lax.axis_index("s")
    base = s * ipw
    b = lax.div(base, L)
    pltpu.sync_copy(act_hbm, vact)
    pltpu.sync_copy(sl_hbm, vsl4)
    lane = lax.iota(jnp.int32, LANES)
    t = lane + lax.rem(base, L)
    bsp = lane * 0 + b
    av0 = plsc.load_gather(vact, [bsp, jnp.minimum(t, L - 2)])
    av1 = plsc.load_gather(vact, [bsp, jnp.minimum(t + 1, L - 2)])
    slb = plsc.load_gather(vsl4, [bsp])
    vfi[...] = b * (L + 1) + (t + 1)
    cpf = pltpu.async_copy(feat_hbm.at[vfi], frow, semf)
    sq0 = (base + lane) * (64 * 64) + lax.div(av0, LN) * 64 \
        + lax.div(lax.rem(av0, LN), GRAN)
    sq1 = (base + lane + 1) * (64 * 64) + lax.div(av1, LN) * 64 \
        + lax.div(lax.rem(av1, LN), GRAN)
    vlm0 = lax.rem(av0, GRAN)
    vlm1 = lax.rem(av1, GRAN)
    is_term = t == slb - 1
    cf_vec = jnp.where(is_term, 1.0, 0.0).astype(jnp.float32)
    cg_vec = jnp.where((t < L - 2) & jnp.logical_not(is_term),
                       GAMMA_, 0.0).astype(jnp.float32)
    vm_vec = jnp.where(t <= L - 2, 1.0, 0.0).astype(jnp.float32)
    offs = [lax.div(jnp.int32(16 * cc) + lane, SUB) * 512
            + lax.rem(jnp.int32(16 * cc) + lane, SUB) * 8
            for cc in range(4)]
    n_pairs = ipw // 2
    for p in range(n_pairs):
        for m in range(2):
            i = 2 * p + m
            sp0 = _splat(sq0, i)
            sp1 = _splat(sq1, i)
            for cc in range(4):
                sl_ = pl.ds(m * 64 + cc * LANES, LANES)
                qp[p, sl_] = sp0 + offs[cc]
                qt[p, sl_] = jnp.minimum(sp1 + offs[cc], n_gran - 1)
    cps = []
    for p in range(n_pairs):
        cps.append((pltpu.async_copy(zpsi_hbm.at[qp.at[p]],
                                     pdst.at[p], semg),
                    pltpu.async_copy(ztgt_hbm.at[qt.at[p]],
                                     tdst.at[p], semg)))
    cpf.wait()
    acc = jnp.zeros((LANES,), jnp.float32)
    for p in range(n_pairs):
        cps[p][0].wait()
        cps[p][1].wait()
        pp = jnp.full((LANES,), p, jnp.int32)
        for m in range(2):
            i = 2 * p + m
            vls = _splat(vlm0, i)
            vlt = _splat(vlm1, i)
            cfs = _splat(cf_vec, i)
            cgs = _splat(cg_vec, i)
            vms = _splat(vm_vec, i)
            for k in range(4):
                rows = lane + (m * 64 + k * LANES)
                pg = plsc.load_gather(pdst, [pp, rows, vls])
                tg = plsc.load_gather(tdst, [pp, rows, vlt])
                fr = frow[i, pl.ds(k * LANES, LANES)]
                d = vms * pg - cgs * tg - cfs * fr
                acc = acc + d * d
    part_ref[...] = acc
    pltpu.sync_copy(part_ref, shared.at[s])
    plsc.subcore_barrier()

    @pl.when(s == 0)
    def _reduce():
        pltpu.sync_copy(shared, gath)
        tot = gath[0, :]
        for i in range(1, N_WORKERS):
            tot = tot + gath[i, :]
        total = _lane_total(tot)
        slv = plsc.load_gather(vsl4, [jnp.minimum(lane, B - 1)])
        slm = jnp.where(lane < B, slv, 0).astype(jnp.float32)
        denom = _lane_total(slm)
        outv[...] = total / denom
        pltpu.sync_copy(outv, out_hbm)


def kernel(psi, target_psi, actions, features, seq_lens):
    B, L, V, F = psi.shape
    n_gran = B * L * (F // SUB) * (V // LN) * SUB * (LN // GRAN)
    ipw = (B * L) // N_WORKERS
    assert (B * L) % N_WORKERS == 0 and L % ipw == 0 and ipw == LANES
    assert F == 64 and V // LN == SUB

    def gran_view(x):
        return (x.reshape(B, L, V // LN, LN, F // SUB, SUB)
                 .transpose(0, 1, 4, 2, 5, 3)
                 .reshape(n_gran, GRAN))

    zpsi = gran_view(psi)
    ztgt = gran_view(target_psi)
    feat2 = features.reshape(B * (L + 1), F)
    act = actions.astype(jnp.int32)
    sl4 = seq_lens.astype(jnp.int32)

    mesh = plsc.VectorSubcoreMesh(
        core_axis_name="c", subcore_axis_name="s", num_cores=1)
    run = pl.kernel(
        functools.partial(_sc_body, (B, L, n_gran, ipw)),
        out_type=jax.ShapeDtypeStruct((LANES,), jnp.float32),
        mesh=mesh,
        compiler_params=pltpu.CompilerParams(
            use_tc_tiling_on_sc=False, needs_layout_passes=False),
        scratch_types=[
            pltpu.VMEM((B, L - 1), jnp.int32),
            pltpu.VMEM((B,), jnp.int32),
            pltpu.VMEM((LANES,), jnp.int32),
            pltpu.VMEM((LANES // 2, 128), jnp.int32),
            pltpu.VMEM((LANES // 2, 128), jnp.int32),
            pltpu.VMEM((LANES // 2, 128, GRAN), jnp.float32),
            pltpu.VMEM((LANES // 2, 128, GRAN), jnp.float32),
            pltpu.VMEM((LANES, F), jnp.float32),
            pltpu.VMEM((LANES,), jnp.float32),
            pltpu.VMEM_SHARED((N_WORKERS, LANES), jnp.float32),
            pltpu.VMEM((N_WORKERS, LANES), jnp.float32),
            pltpu.VMEM((LANES,), jnp.float32),
            pltpu.SemaphoreType.DMA,
            pltpu.SemaphoreType.DMA,
        ],
    )
    out = run(zpsi, ztgt, feat2, act, sl4)
    return out[0]


# speedup vs baseline: 1.1690x; 1.0373x over previous
import functools

import jax
import jax.numpy as jnp
from jax import lax
from jax.experimental import pallas as pl
from jax.experimental.pallas import tpu as pltpu
from jax.experimental.pallas import tpu_sc as plsc

GAMMA_ = 0.99
LANES = 16
N_WORKERS = 16
SUB = 8
LN = 128
GRAN = 16


def _splat(vec, i):
    ci = jnp.full((LANES,), i, jnp.int32)
    return vec.at[ci].get(mode="promise_in_bounds")


def _lane_total(vec):
    return _splat(plsc.cumsum(vec), LANES - 1)


def _sc_body(consts, zpsi_hbm, ztgt_hbm, feat_hbm, act_hbm, sl_hbm,
             out_hbm, vact, vsl4, vfi, qp, qt, pdst, tdst, frow, part_ref,
             shared, gath, outv, semf, semg):
    B, L, n_gran, ipw = consts
    s = lax.axis_index("s")
    base = s * ipw
    b = lax.div(base, L)
    lane = lax.iota(jnp.int32, LANES)
    t = lane + lax.rem(base, L)
    vfi[...] = b * (L + 1) + (t + 1)
    cpf = pltpu.async_copy(feat_hbm.at[vfi], frow, semf)
    cpa = pltpu.async_copy(act_hbm, vact, semf)
    cpl = pltpu.async_copy(sl_hbm, vsl4, semf)
    cpa.wait()
    cpl.wait()
    bsp = lane * 0 + b
    av0 = plsc.load_gather(vact, [bsp, jnp.minimum(t, L - 2)])
    av1 = plsc.load_gather(vact, [bsp, jnp.minimum(t + 1, L - 2)])
    slb = plsc.load_gather(vsl4, [bsp])
    sq0 = (base + lane) * (64 * 64) + lax.div(av0, LN) * 64 \
        + lax.div(lax.rem(av0, LN), GRAN)
    sq1 = (base + lane + 1) * (64 * 64) + lax.div(av1, LN) * 64 \
        + lax.div(lax.rem(av1, LN), GRAN)
    vlm0 = lax.rem(av0, GRAN)
    vlm1 = lax.rem(av1, GRAN)
    is_term = t == slb - 1
    cf_vec = jnp.where(is_term, 1.0, 0.0).astype(jnp.float32)
    cg_vec = jnp.where((t < L - 2) & jnp.logical_not(is_term),
                       GAMMA_, 0.0).astype(jnp.float32)
    vm_vec = jnp.where(t <= L - 2, 1.0, 0.0).astype(jnp.float32)
    offs = [lax.div(jnp.int32(16 * cc) + lane, SUB) * 512
            + lax.rem(jnp.int32(16 * cc) + lane, SUB) * 8
            for cc in range(4)]
    n_pairs = ipw // 2
    for p in range(n_pairs):
        for m in range(2):
            i = 2 * p + m
            sp0 = _splat(sq0, i)
            sp1 = _splat(sq1, i)
            for cc in range(4):
                sl_ = pl.ds(m * 64 + cc * LANES, LANES)
                qp[p, sl_] = sp0 + offs[cc]
                qt[p, sl_] = jnp.minimum(sp1 + offs[cc], n_gran - 1)
    cps = []
    for p in range(n_pairs):
        cps.append((pltpu.async_copy(zpsi_hbm.at[qp.at[p]],
                                     pdst.at[p], semg),
                    pltpu.async_copy(ztgt_hbm.at[qt.at[p]],
                                     tdst.at[p], semg)))
    cpf.wait()
    acc = jnp.zeros((LANES,), jnp.float32)
    for p in range(n_pairs):
        cps[p][0].wait()
        cps[p][1].wait()
        pp = jnp.full((LANES,), p, jnp.int32)
        for m in range(2):
            i = 2 * p + m
            vls = _splat(vlm0, i)
            vlt = _splat(vlm1, i)
            cfs = _splat(cf_vec, i)
            cgs = _splat(cg_vec, i)
            vms = _splat(vm_vec, i)
            for k in range(4):
                rows = lane + (m * 64 + k * LANES)
                pg = plsc.load_gather(pdst, [pp, rows, vls])
                tg = plsc.load_gather(tdst, [pp, rows, vlt])
                fr = frow[i, pl.ds(k * LANES, LANES)]
                d = vms * pg - cgs * tg - cfs * fr
                acc = acc + d * d
    part_ref[...] = acc
    pltpu.sync_copy(part_ref, shared.at[s])
    plsc.subcore_barrier()

    @pl.when(s == 0)
    def _reduce():
        pltpu.sync_copy(shared, gath)
        tot = gath[0, :]
        for i in range(1, N_WORKERS):
            tot = tot + gath[i, :]
        total = _lane_total(tot)
        slv = plsc.load_gather(vsl4, [jnp.minimum(lane, B - 1)])
        slm = jnp.where(lane < B, slv, 0).astype(jnp.float32)
        denom = _lane_total(slm)
        outv[...] = total / denom
        pltpu.sync_copy(outv, out_hbm)


def kernel(psi, target_psi, actions, features, seq_lens):
    B, L, V, F = psi.shape
    n_gran = B * L * (F // SUB) * (V // LN) * SUB * (LN // GRAN)
    ipw = (B * L) // N_WORKERS
    assert (B * L) % N_WORKERS == 0 and L % ipw == 0 and ipw == LANES
    assert F == 64 and V // LN == SUB

    def gran_view(x):
        return (x.reshape(B, L, V // LN, LN, F // SUB, SUB)
                 .transpose(0, 1, 4, 2, 5, 3)
                 .reshape(n_gran, GRAN))

    zpsi = gran_view(psi)
    ztgt = gran_view(target_psi)
    feat2 = features.reshape(B * (L + 1), F)
    act = actions.astype(jnp.int32)
    sl4 = seq_lens.astype(jnp.int32)

    mesh = plsc.VectorSubcoreMesh(
        core_axis_name="c", subcore_axis_name="s", num_cores=1)
    run = pl.kernel(
        functools.partial(_sc_body, (B, L, n_gran, ipw)),
        out_type=jax.ShapeDtypeStruct((LANES,), jnp.float32),
        mesh=mesh,
        compiler_params=pltpu.CompilerParams(
            use_tc_tiling_on_sc=False, needs_layout_passes=False),
        scratch_types=[
            pltpu.VMEM((B, L - 1), jnp.int32),
            pltpu.VMEM((B,), jnp.int32),
            pltpu.VMEM((LANES,), jnp.int32),
            pltpu.VMEM((LANES // 2, 128), jnp.int32),
            pltpu.VMEM((LANES // 2, 128), jnp.int32),
            pltpu.VMEM((LANES // 2, 128, GRAN), jnp.float32),
            pltpu.VMEM((LANES // 2, 128, GRAN), jnp.float32),
            pltpu.VMEM((LANES, F), jnp.float32),
            pltpu.VMEM((LANES,), jnp.float32),
            pltpu.VMEM_SHARED((N_WORKERS, LANES), jnp.float32),
            pltpu.VMEM((N_WORKERS, LANES), jnp.float32),
            pltpu.VMEM((LANES,), jnp.float32),
            pltpu.SemaphoreType.DMA,
            pltpu.SemaphoreType.DMA,
        ],
    )
    out = run(zpsi, ztgt, feat2, act, sl4)
    return out[0]


# speedup vs baseline: 1.1758x; 1.0058x over previous
import functools

import jax
import jax.numpy as jnp
from jax import lax
from jax.experimental import pallas as pl
from jax.experimental.pallas import tpu as pltpu
from jax.experimental.pallas import tpu_sc as plsc

GAMMA_ = 0.99
LANES = 16
N_WORKERS = 16
SUB = 8
LN = 128
GRAN = 16


def _splat(vec, i):
    ci = jnp.full((LANES,), i, jnp.int32)
    return vec.at[ci].get(mode="promise_in_bounds")


def _lane_total(vec):
    return _splat(plsc.cumsum(vec), LANES - 1)


def _sc_body(consts, zpsi_hbm, ztgt_hbm, feat_hbm, act_hbm, sl_hbm,
             out_hbm, vact, vsl4, vfi, qp, qt, pdst, tdst, frow, part_ref,
             shared, gath, outv, semf, sema, seml, semg):
    B, L, n_gran, ipw = consts
    s = lax.axis_index("s")
    base = s * ipw
    b = lax.div(base, L)
    lane = lax.iota(jnp.int32, LANES)
    t = lane + lax.rem(base, L)
    vfi[...] = b * (L + 1) + (t + 1)
    cpf = pltpu.async_copy(feat_hbm.at[vfi], frow, semf)
    cpa = pltpu.async_copy(act_hbm, vact, sema)
    cpl = pltpu.async_copy(sl_hbm, vsl4, seml)
    cpa.wait()
    cpl.wait()
    bsp = lane * 0 + b
    av0 = plsc.load_gather(vact, [bsp, jnp.minimum(t, L - 2)])
    av1 = plsc.load_gather(vact, [bsp, jnp.minimum(t + 1, L - 2)])
    slb = plsc.load_gather(vsl4, [bsp])
    sq0 = (base + lane) * (64 * 64) + lax.div(av0, LN) * 64 \
        + lax.div(lax.rem(av0, LN), GRAN)
    sq1 = (base + lane + 1) * (64 * 64) + lax.div(av1, LN) * 64 \
        + lax.div(lax.rem(av1, LN), GRAN)
    vlm0 = lax.rem(av0, GRAN)
    vlm1 = lax.rem(av1, GRAN)
    is_term = t == slb - 1
    cf_vec = jnp.where(is_term, 1.0, 0.0).astype(jnp.float32)
    cg_vec = jnp.where((t < L - 2) & jnp.logical_not(is_term),
                       GAMMA_, 0.0).astype(jnp.float32)
    vm_vec = jnp.where(t <= L - 2, 1.0, 0.0).astype(jnp.float32)
    offs = [lax.div(jnp.int32(16 * cc) + lane, SUB) * 512
            + lax.rem(jnp.int32(16 * cc) + lane, SUB) * 8
            for cc in range(4)]
    n_pairs = ipw // 2
    for p in range(n_pairs):
        for m in range(2):
            i = 2 * p + m
            sp0 = _splat(sq0, i)
            sp1 = _splat(sq1, i)
            for cc in range(4):
                sl_ = pl.ds(m * 64 + cc * LANES, LANES)
                qp[p, sl_] = sp0 + offs[cc]
                qt[p, sl_] = jnp.minimum(sp1 + offs[cc], n_gran - 1)
    cps = []
    for p in range(n_pairs):
        cps.append((pltpu.async_copy(zpsi_hbm.at[qp.at[p]],
                                     pdst.at[p], semg),
                    pltpu.async_copy(ztgt_hbm.at[qt.at[p]],
                                     tdst.at[p], semg)))
    cpf.wait()
    acc = jnp.zeros((LANES,), jnp.float32)
    for p in range(n_pairs):
        cps[p][0].wait()
        cps[p][1].wait()
        pp = jnp.full((LANES,), p, jnp.int32)
        for m in range(2):
            i = 2 * p + m
            vls = _splat(vlm0, i)
            vlt = _splat(vlm1, i)
            cfs = _splat(cf_vec, i)
            cgs = _splat(cg_vec, i)
            vms = _splat(vm_vec, i)
            for k in range(4):
                rows = lane + (m * 64 + k * LANES)
                pg = plsc.load_gather(pdst, [pp, rows, vls])
                tg = plsc.load_gather(tdst, [pp, rows, vlt])
                fr = frow[i, pl.ds(k * LANES, LANES)]
                d = vms * pg - cgs * tg - cfs * fr
                acc = acc + d * d
    part_ref[...] = acc
    pltpu.sync_copy(part_ref, shared.at[s])
    plsc.subcore_barrier()

    @pl.when(s == 0)
    def _reduce():
        pltpu.sync_copy(shared, gath)
        tot = gath[0, :]
        for i in range(1, N_WORKERS):
            tot = tot + gath[i, :]
        total = _lane_total(tot)
        slv = plsc.load_gather(vsl4, [jnp.minimum(lane, B - 1)])
        slm = jnp.where(lane < B, slv, 0).astype(jnp.float32)
        denom = _lane_total(slm)
        outv[...] = total / denom
        pltpu.sync_copy(outv, out_hbm)


def kernel(psi, target_psi, actions, features, seq_lens):
    B, L, V, F = psi.shape
    n_gran = B * L * (F // SUB) * (V // LN) * SUB * (LN // GRAN)
    ipw = (B * L) // N_WORKERS
    assert (B * L) % N_WORKERS == 0 and L % ipw == 0 and ipw == LANES
    assert F == 64 and V // LN == SUB

    def gran_view(x):
        return (x.reshape(B, L, V // LN, LN, F // SUB, SUB)
                 .transpose(0, 1, 4, 2, 5, 3)
                 .reshape(n_gran, GRAN))

    zpsi = gran_view(psi)
    ztgt = gran_view(target_psi)
    feat2 = features.reshape(B * (L + 1), F)
    act = actions.astype(jnp.int32)
    sl4 = seq_lens.astype(jnp.int32)

    mesh = plsc.VectorSubcoreMesh(
        core_axis_name="c", subcore_axis_name="s", num_cores=1)
    run = pl.kernel(
        functools.partial(_sc_body, (B, L, n_gran, ipw)),
        out_type=jax.ShapeDtypeStruct((LANES,), jnp.float32),
        mesh=mesh,
        compiler_params=pltpu.CompilerParams(
            use_tc_tiling_on_sc=False, needs_layout_passes=False),
        scratch_types=[
            pltpu.VMEM((B, L - 1), jnp.int32),
            pltpu.VMEM((B,), jnp.int32),
            pltpu.VMEM((LANES,), jnp.int32),
            pltpu.VMEM((LANES // 2, 128), jnp.int32),
            pltpu.VMEM((LANES // 2, 128), jnp.int32),
            pltpu.VMEM((LANES // 2, 128, GRAN), jnp.float32),
            pltpu.VMEM((LANES // 2, 128, GRAN), jnp.float32),
            pltpu.VMEM((LANES, F), jnp.float32),
            pltpu.VMEM((LANES,), jnp.float32),
            pltpu.VMEM_SHARED((N_WORKERS, LANES), jnp.float32),
            pltpu.VMEM((N_WORKERS, LANES), jnp.float32),
            pltpu.VMEM((LANES,), jnp.float32),
            pltpu.SemaphoreType.DMA,
            pltpu.SemaphoreType.DMA,
            pltpu.SemaphoreType.DMA,
            pltpu.SemaphoreType.DMA,
        ],
    )
    out = run(zpsi, ztgt, feat2, act, sl4)
    return out[0]
